# baseline (device time: 331465 ns/iter reference)
import jax
import jax.numpy as jnp
from jax import lax
from jax.experimental import pallas as pl
from jax.experimental.pallas import tpu as pltpu

W = 32
M = 1536
N = 1536
CH = M // W


def kernel(A, B):
    m, k = A.shape
    _, n = B.shape

    def body(a_ref, b_ref, out_ref, comm_ref, send_sems, recv_sems):
        my = lax.axis_index("i")
        left = lax.rem(my + (W - 1), W)
        right = lax.rem(my + 1, W)

        barrier = pltpu.get_barrier_semaphore()
        pl.semaphore_signal(barrier, inc=1, device_id=(left,),
                            device_id_type=pl.DeviceIdType.MESH)
        pl.semaphore_signal(barrier, inc=1, device_id=(right,),
                            device_id_type=pl.DeviceIdType.MESH)
        pl.semaphore_wait(barrier, 2)

        out_ref[...] = jnp.dot(a_ref[...], b_ref[...],
                               preferred_element_type=jnp.float32)

        for s in range(W - 1):
            send_idx = lax.rem(my - s + W, W)
            recv_idx = lax.rem(my - s - 1 + 2 * W, W)
            rdma = pltpu.make_async_remote_copy(
                src_ref=out_ref.at[pl.ds(send_idx * CH, CH), :],
                dst_ref=comm_ref.at[s],
                send_sem=send_sems.at[s],
                recv_sem=recv_sems.at[s],
                device_id=(right,),
                device_id_type=pl.DeviceIdType.MESH,
            )
            rdma.start()
            rdma.wait()
            out_ref[pl.ds(recv_idx * CH, CH), :] = (
                out_ref[pl.ds(recv_idx * CH, CH), :] + comm_ref[s]
            )

        for s in range(W - 1):
            send_idx = lax.rem(my + 1 - s + 2 * W, W)
            rdma = pltpu.make_async_remote_copy(
                src_ref=out_ref.at[pl.ds(send_idx * CH, CH), :],
                dst_ref=out_ref.at[pl.ds(send_idx * CH, CH), :],
                send_sem=send_sems.at[(W - 1) + s],
                recv_sem=recv_sems.at[(W - 1) + s],
                device_id=(right,),
                device_id_type=pl.DeviceIdType.MESH,
            )
            rdma.start()
            rdma.wait()

        z = out_ref[...]
        out_ref[...] = 0.5 * z * (
            1.0 + jnp.tanh(0.7978845608 * (z + 0.044715 * z * z * z))
        )

    return pl.pallas_call(
        body,
        out_shape=jax.ShapeDtypeStruct((M, N), jnp.float32),
        in_specs=[
            pl.BlockSpec(memory_space=pltpu.VMEM),
            pl.BlockSpec(memory_space=pltpu.VMEM),
        ],
        out_specs=pl.BlockSpec(memory_space=pltpu.VMEM),
        scratch_shapes=[
            pltpu.VMEM((W - 1, CH, N), jnp.float32),
            pltpu.SemaphoreType.DMA((2 * (W - 1),)),
            pltpu.SemaphoreType.DMA((2 * (W - 1),)),
        ],
        compiler_params=pltpu.CompilerParams(collective_id=0),
    )(A, B)
